# baseline (speedup 1.0000x reference)
import jax
import jax.numpy as jnp
from jax.experimental import pallas as pl
from jax.experimental.pallas import tpu as pltpu


def _fused_body(x_ref, w_ref, a_ref, b_ref, out_ref, s_ref):
    @pl.when(pl.program_id(0) == 0)
    def _():
        s_ref[...] = jnp.dot(
            x_ref[...].astype(jnp.bfloat16), w_ref[...].astype(jnp.bfloat16),
            preferred_element_type=jnp.float32).astype(jnp.bfloat16)

    acc = jax.lax.dot_general(
        a_ref[...], s_ref[...], (((1,), (0,)), ((), ())),
        precision=jax.lax.Precision.DEFAULT,
        preferred_element_type=jnp.float32)
    out_ref[...] = acc + b_ref[...]


def kernel(input, adj, weight, bias):
    x = jnp.squeeze(input)
    a = jnp.squeeze(adj)
    n, f_in = x.shape
    f_out = weight.shape[-1]
    bias2d = bias.reshape(1, f_out)

    bm = 128
    grid_m = pl.cdiv(n, bm)
    out = pl.pallas_call(
        _fused_body,
        grid=(grid_m,),
        in_specs=[
            pl.BlockSpec((n, f_in), lambda i: (0, 0)),
            pl.BlockSpec((f_in, f_out), lambda i: (0, 0)),
            pl.BlockSpec((bm, n), lambda i: (i, 0)),
            pl.BlockSpec((1, f_out), lambda i: (0, 0)),
        ],
        out_specs=pl.BlockSpec((bm, f_out), lambda i: (i, 0)),
        out_shape=jax.ShapeDtypeStruct((n, f_out), jnp.float32),
        scratch_shapes=[pltpu.VMEM((n, f_out), jnp.bfloat16)],
        compiler_params=pltpu.CompilerParams(
            dimension_semantics=("arbitrary",),
        ),
    )(x, weight, a, bias2d)
    return out


# speedup vs baseline: 1.1286x; 1.1286x over previous
import jax
import jax.numpy as jnp
from jax.experimental import pallas as pl
from jax.experimental.pallas import tpu as pltpu


def _dot(a, b):
    return jax.lax.dot_general(
        a, b, (((1,), (0,)), ((), ())),
        precision=jax.lax.Precision.DEFAULT,
        preferred_element_type=jnp.float32)


def _fused_body(x_ref, w_ref, a_ref, b_ref, out_ref, s_ref):
    @pl.when(pl.program_id(0) == 0)
    def _():
        s_ref[...] = _dot(x_ref[...], w_ref[...])

    out_ref[...] = _dot(a_ref[...], s_ref[...]) + b_ref[...]


def kernel(input, adj, weight, bias):
    x = jnp.squeeze(input)
    a = jnp.squeeze(adj)
    n, f_in = x.shape
    f_out = weight.shape[-1]
    bias2d = bias.reshape(1, f_out)

    bm = 256
    grid_m = pl.cdiv(n, bm)
    out = pl.pallas_call(
        _fused_body,
        grid=(grid_m,),
        in_specs=[
            pl.BlockSpec((n, f_in), lambda i: (0, 0)),
            pl.BlockSpec((f_in, f_out), lambda i: (0, 0)),
            pl.BlockSpec((bm, n), lambda i: (i, 0)),
            pl.BlockSpec((1, f_out), lambda i: (0, 0)),
        ],
        out_specs=pl.BlockSpec((bm, f_out), lambda i: (i, 0)),
        out_shape=jax.ShapeDtypeStruct((n, f_out), jnp.float32),
        scratch_shapes=[pltpu.VMEM((n, f_out), jnp.float32)],
        compiler_params=pltpu.CompilerParams(
            dimension_semantics=("arbitrary",),
        ),
    )(x, weight, a, bias2d)
    return out
